# baseline (device time: 20446 ns/iter reference)
import os

import jax
import jax.numpy as jnp
from jax import lax
from jax.experimental import pallas as pl
from jax.experimental.pallas import tpu as pltpu

ABLATE = int(os.environ.get("ABLATE", "0"))
N_Z = 4
QM = 128
WAVES = 4
HM = QM // WAVES


def kernel(dy, W):
    m, k = dy.shape
    n = W.shape[0]

    def body(dy_hbm, w_hbm, out_hbm, dyq_ref, w_ref, acc_ref, psum_ref,
             oq_ref, a16_ref, p16_ref, o16_ref, zbuf_ref, xybuf_ref,
             ostage_ref, in_sems, send_sems, zrecv_sems,
             xyrecv_sems, ostore_sems, zr2_sem, xy_sem):
        my_x = lax.axis_index("x")
        my_y = lax.axis_index("y")
        my_z = lax.axis_index("z")
        c = 2 * my_x + my_y

        is_z0 = my_z == 0
        is_z1 = my_z == 1
        is_z3 = my_z == N_Z - 1
        is_edge = jnp.logical_or(is_z0, is_z3)
        is_mid = jnp.logical_not(is_edge)

        pair_z = my_z + jnp.where(jnp.logical_or(is_z0, my_z == 2), 1, -1)
        other_mid = jnp.where(is_z1, 2, 1)
        far_edge = jnp.where(is_z1, 3, 0)
        writer_mid = jnp.where(my_z <= 1, 2, 1)

        def rcopy(src, dst, ssem, rsem, dev):
            return pltpu.make_async_remote_copy(
                src_ref=src, dst_ref=dst, send_sem=ssem, recv_sem=rsem,
                device_id=dev, device_id_type=pl.DeviceIdType.MESH,
            )

        dcp = pltpu.make_async_copy(
            dy_hbm.at[pl.ds(c * QM, QM), :], dyq_ref, in_sems.at[0])
        wcp = pltpu.make_async_copy(w_hbm, w_ref, in_sems.at[1])
        dcp.start()
        wcp.start()

        barrier_sem = pltpu.get_barrier_semaphore()
        pl.semaphore_signal(
            barrier_sem, inc=1, device_id=(my_x, my_y, pair_z),
            device_id_type=pl.DeviceIdType.MESH,
        )
        if ABLATE != 1:
            pl.semaphore_signal(
                zr2_sem, inc=1, device_id=(my_x, my_y, writer_mid),
                device_id_type=pl.DeviceIdType.MESH,
            )
        xy_devs = (
            (1 - my_x, 1 - my_y, my_z),
            (1 - my_x, my_y, my_z),
            (my_x, 1 - my_y, my_z),
        )
        for dev in xy_devs:
            pl.semaphore_signal(
                xy_sem, inc=1, device_id=dev,
                device_id_type=pl.DeviceIdType.MESH,
            )

        dcp.wait()
        wcp.wait()

        for h in range(WAVES):
            acc_ref[h] = lax.dot_general(
                dyq_ref[pl.ds(h * HM, HM), :],
                w_ref[...],
                dimension_numbers=(((1,), (1,)), ((), ())),
                preferred_element_type=jnp.float32,
            )
            a16_ref[h] = acc_ref[h].astype(jnp.bfloat16)
            if h == 0:
                pl.semaphore_wait(barrier_sem, 1)
            if ABLATE != 1:
                rcopy(a16_ref.at[h], zbuf_ref.at[h, 0], send_sems.at[h, 0],
                      zrecv_sems.at[h, 0], (my_x, my_y, pair_z)).start()

        for h in range(WAVES) if ABLATE != 1 else ():
            rcopy(a16_ref.at[h], zbuf_ref.at[h, 0], send_sems.at[h, 0],
                  zrecv_sems.at[h, 0], (my_x, my_y, pair_z)).wait()
            psum_ref[h] = acc_ref[h] + zbuf_ref[h, 0].astype(jnp.float32)
            p16_ref[h] = psum_ref[h].astype(jnp.bfloat16)

            @pl.when(is_mid)
            def _(h=h):
                if h == 0:
                    pl.semaphore_wait(zr2_sem, 2)
                rcopy(p16_ref.at[h], zbuf_ref.at[h, 1], send_sems.at[h, 2],
                      zrecv_sems.at[h, 1], (my_x, my_y, far_edge)).start()
                rcopy(p16_ref.at[h], zbuf_ref.at[h, 1], send_sems.at[h, 1],
                      zrecv_sems.at[h, 1], (my_x, my_y, other_mid)).start()

        ostores = []
        for h in range(WAVES):
            if ABLATE == 1:
                oq_ref[h] = acc_ref[h]
            else:
                rcopy(p16_ref.at[h], zbuf_ref.at[h, 1], send_sems.at[h, 1],
                      zrecv_sems.at[h, 1], (my_x, my_y, pair_z)).wait_recv()
                oq_ref[h] = psum_ref[h] + zbuf_ref[h, 1].astype(jnp.float32)
            o16_ref[h] = oq_ref[h].astype(jnp.bfloat16)
            if h == 0:
                pl.semaphore_wait(xy_sem, 3)
            if ABLATE == 0:
                for j, dev in enumerate(xy_devs):
                    rcopy(o16_ref.at[h], xybuf_ref.at[h, j],
                          send_sems.at[h, 3 + j],
                          xyrecv_sems.at[h, j], dev).start()
            st = pltpu.make_async_copy(
                oq_ref.at[h],
                out_hbm.at[pl.ds(c * QM + h * HM, HM), :],
                ostore_sems.at[h, 0])
            st.start()
            ostores.append(st)

        src_cols = (
            2 * (1 - my_x) + (1 - my_y),
            2 * (1 - my_x) + my_y,
            2 * my_x + (1 - my_y),
        )
        for h in range(WAVES):
            for j, (dev, cc) in enumerate(zip(xy_devs, src_cols)):
                if ABLATE == 0:
                    rcopy(o16_ref.at[h], xybuf_ref.at[h, j],
                          send_sems.at[h, 3 + j],
                          xyrecv_sems.at[h, j], dev).wait_recv()
                    ostage_ref[h, j] = xybuf_ref[h, j].astype(jnp.float32)
                src = ostage_ref.at[h, j] if ABLATE == 0 else oq_ref.at[h]
                st = pltpu.make_async_copy(
                    src,
                    out_hbm.at[pl.ds(cc * QM + h * HM, HM), :],
                    ostore_sems.at[h, 1 + j])
                st.start()
                ostores.append(st)

        for st in ostores:
            st.wait()
        for h in range(WAVES) if ABLATE != 1 else ():
            if ABLATE == 0:
                for j, dev in enumerate(xy_devs):
                    rcopy(o16_ref.at[h], xybuf_ref.at[h, j],
                          send_sems.at[h, 3 + j],
                          xyrecv_sems.at[h, j], dev).wait_send()

            @pl.when(is_mid)
            def _(h=h):
                rcopy(p16_ref.at[h], zbuf_ref.at[h, 1], send_sems.at[h, 1],
                      zrecv_sems.at[h, 1], (my_x, my_y, other_mid)).wait_send()
                rcopy(p16_ref.at[h], zbuf_ref.at[h, 1], send_sems.at[h, 2],
                      zrecv_sems.at[h, 1], (my_x, my_y, far_edge)).wait_send()

    return pl.pallas_call(
        body,
        out_shape=jax.ShapeDtypeStruct((m, n), jnp.float32),
        in_specs=[
            pl.BlockSpec(memory_space=pl.ANY),
            pl.BlockSpec(memory_space=pl.ANY),
        ],
        out_specs=pl.BlockSpec(memory_space=pl.ANY),
        scratch_shapes=[
            pltpu.VMEM((QM, k), jnp.float32),
            pltpu.VMEM((n, k), jnp.float32),
            pltpu.VMEM((WAVES, HM, n), jnp.float32),
            pltpu.VMEM((WAVES, HM, n), jnp.float32),
            pltpu.VMEM((WAVES, HM, n), jnp.float32),
            pltpu.VMEM((WAVES, HM, n), jnp.bfloat16),
            pltpu.VMEM((WAVES, HM, n), jnp.bfloat16),
            pltpu.VMEM((WAVES, HM, n), jnp.bfloat16),
            pltpu.VMEM((WAVES, 2, HM, n), jnp.bfloat16),
            pltpu.VMEM((WAVES, 3, HM, n), jnp.bfloat16),
            pltpu.VMEM((WAVES, 3, HM, n), jnp.float32),
            pltpu.SemaphoreType.DMA((2,)),
            pltpu.SemaphoreType.DMA((WAVES, 6)),
            pltpu.SemaphoreType.DMA((WAVES, 2)),
            pltpu.SemaphoreType.DMA((WAVES, 3)),
            pltpu.SemaphoreType.DMA((WAVES, 4)),
            pltpu.SemaphoreType.REGULAR,
            pltpu.SemaphoreType.REGULAR,
        ],
        compiler_params=pltpu.CompilerParams(collective_id=0),
    )(dy, W)


# device time: 20109 ns/iter; 1.0168x vs baseline; 1.0168x over previous
import os

import jax
import jax.numpy as jnp
from jax import lax
from jax.experimental import pallas as pl
from jax.experimental.pallas import tpu as pltpu

ABLATE = int(os.environ.get("ABLATE", "0"))
N_Z = 4
QM = 128
WAVES = 4
HM = QM // WAVES


def kernel(dy, W):
    m, k = dy.shape
    n = W.shape[0]

    def body(dy_ref, w_ref, out_hbm, acc_ref, psum_ref,
             oq_ref, a16_ref, p16_ref, o16_ref, zbuf_ref, xybuf_ref,
             ostage_ref, send_sems, zrecv_sems,
             xyrecv_sems, ostore_sems, zr2_sem, xy_sem):
        my_x = lax.axis_index("x")
        my_y = lax.axis_index("y")
        my_z = lax.axis_index("z")
        c = 2 * my_x + my_y

        is_z0 = my_z == 0
        is_z1 = my_z == 1
        is_z3 = my_z == N_Z - 1
        is_edge = jnp.logical_or(is_z0, is_z3)
        is_mid = jnp.logical_not(is_edge)

        pair_z = my_z + jnp.where(jnp.logical_or(is_z0, my_z == 2), 1, -1)
        other_mid = jnp.where(is_z1, 2, 1)
        far_edge = jnp.where(is_z1, 3, 0)
        writer_mid = jnp.where(my_z <= 1, 2, 1)

        def rcopy(src, dst, ssem, rsem, dev):
            return pltpu.make_async_remote_copy(
                src_ref=src, dst_ref=dst, send_sem=ssem, recv_sem=rsem,
                device_id=dev, device_id_type=pl.DeviceIdType.MESH,
            )

        barrier_sem = pltpu.get_barrier_semaphore()
        pl.semaphore_signal(
            barrier_sem, inc=1, device_id=(my_x, my_y, pair_z),
            device_id_type=pl.DeviceIdType.MESH,
        )
        if ABLATE not in (1, 3):
            pl.semaphore_signal(
                zr2_sem, inc=1, device_id=(my_x, my_y, writer_mid),
                device_id_type=pl.DeviceIdType.MESH,
            )
        xy_devs = (
            (1 - my_x, 1 - my_y, my_z),
            (1 - my_x, my_y, my_z),
            (my_x, 1 - my_y, my_z),
        )
        for dev in xy_devs:
            pl.semaphore_signal(
                xy_sem, inc=1, device_id=dev,
                device_id_type=pl.DeviceIdType.MESH,
            )

        for h in range(WAVES):
            if ABLATE == 3:
                acc_ref[h] = jnp.zeros((HM, n), jnp.float32)
            else:
                acc_ref[h] = lax.dot_general(
                    dy_ref[pl.ds(c * QM + h * HM, HM), :],
                    w_ref[...],
                    dimension_numbers=(((1,), (1,)), ((), ())),
                    preferred_element_type=jnp.float32,
                )
            a16_ref[h] = acc_ref[h].astype(jnp.bfloat16)
            if h == 0:
                pl.semaphore_wait(barrier_sem, 1)
            if ABLATE not in (1, 3):
                rcopy(a16_ref.at[h], zbuf_ref.at[h, 0], send_sems.at[h, 0],
                      zrecv_sems.at[h, 0], (my_x, my_y, pair_z)).start()

        for h in range(WAVES) if ABLATE not in (1, 3) else ():
            rcopy(a16_ref.at[h], zbuf_ref.at[h, 0], send_sems.at[h, 0],
                  zrecv_sems.at[h, 0], (my_x, my_y, pair_z)).wait()
            psum_ref[h] = acc_ref[h] + zbuf_ref[h, 0].astype(jnp.float32)
            p16_ref[h] = psum_ref[h].astype(jnp.bfloat16)

            @pl.when(is_mid)
            def _(h=h):
                if h == 0:
                    pl.semaphore_wait(zr2_sem, 2)
                rcopy(p16_ref.at[h], zbuf_ref.at[h, 1], send_sems.at[h, 2],
                      zrecv_sems.at[h, 1], (my_x, my_y, far_edge)).start()
                rcopy(p16_ref.at[h], zbuf_ref.at[h, 1], send_sems.at[h, 1],
                      zrecv_sems.at[h, 1], (my_x, my_y, other_mid)).start()

        ostores = []
        for h in range(WAVES):
            if ABLATE in (1, 3):
                oq_ref[h] = acc_ref[h]
            else:
                rcopy(p16_ref.at[h], zbuf_ref.at[h, 1], send_sems.at[h, 1],
                      zrecv_sems.at[h, 1], (my_x, my_y, pair_z)).wait_recv()
                oq_ref[h] = psum_ref[h] + zbuf_ref[h, 1].astype(jnp.float32)
            o16_ref[h] = oq_ref[h].astype(jnp.bfloat16)
            if h == 0:
                pl.semaphore_wait(xy_sem, 3)
            if ABLATE == 0:
                for j, dev in enumerate(xy_devs):
                    rcopy(o16_ref.at[h], xybuf_ref.at[h, j],
                          send_sems.at[h, 3 + j],
                          xyrecv_sems.at[h, j], dev).start()
            st = pltpu.make_async_copy(
                oq_ref.at[h],
                out_hbm.at[pl.ds(c * QM + h * HM, HM), :],
                ostore_sems.at[h, 0])
            st.start()
            ostores.append(st)

        src_cols = (
            2 * (1 - my_x) + (1 - my_y),
            2 * (1 - my_x) + my_y,
            2 * my_x + (1 - my_y),
        )
        for h in range(WAVES):
            for j, (dev, cc) in enumerate(zip(xy_devs, src_cols)):
                if ABLATE == 0:
                    rcopy(o16_ref.at[h], xybuf_ref.at[h, j],
                          send_sems.at[h, 3 + j],
                          xyrecv_sems.at[h, j], dev).wait_recv()
                    ostage_ref[h, j] = xybuf_ref[h, j].astype(jnp.float32)
                src = ostage_ref.at[h, j] if ABLATE == 0 else oq_ref.at[h]
                st = pltpu.make_async_copy(
                    src,
                    out_hbm.at[pl.ds(cc * QM + h * HM, HM), :],
                    ostore_sems.at[h, 1 + j])
                st.start()
                ostores.append(st)

        for st in ostores:
            st.wait()
        for h in range(WAVES) if ABLATE not in (1, 3) else ():
            if ABLATE == 0:
                for j, dev in enumerate(xy_devs):
                    rcopy(o16_ref.at[h], xybuf_ref.at[h, j],
                          send_sems.at[h, 3 + j],
                          xyrecv_sems.at[h, j], dev).wait_send()

            @pl.when(is_mid)
            def _(h=h):
                rcopy(p16_ref.at[h], zbuf_ref.at[h, 1], send_sems.at[h, 1],
                      zrecv_sems.at[h, 1], (my_x, my_y, other_mid)).wait_send()
                rcopy(p16_ref.at[h], zbuf_ref.at[h, 1], send_sems.at[h, 2],
                      zrecv_sems.at[h, 1], (my_x, my_y, far_edge)).wait_send()

    return pl.pallas_call(
        body,
        out_shape=jax.ShapeDtypeStruct((m, n), jnp.float32),
        in_specs=[
            pl.BlockSpec(memory_space=pltpu.VMEM),
            pl.BlockSpec(memory_space=pltpu.VMEM),
        ],
        out_specs=pl.BlockSpec(memory_space=pl.ANY),
        scratch_shapes=[
            pltpu.VMEM((WAVES, HM, n), jnp.float32),
            pltpu.VMEM((WAVES, HM, n), jnp.float32),
            pltpu.VMEM((WAVES, HM, n), jnp.float32),
            pltpu.VMEM((WAVES, HM, n), jnp.bfloat16),
            pltpu.VMEM((WAVES, HM, n), jnp.bfloat16),
            pltpu.VMEM((WAVES, HM, n), jnp.bfloat16),
            pltpu.VMEM((WAVES, 2, HM, n), jnp.bfloat16),
            pltpu.VMEM((WAVES, 3, HM, n), jnp.bfloat16),
            pltpu.VMEM((WAVES, 3, HM, n), jnp.float32),
            pltpu.SemaphoreType.DMA((WAVES, 6)),
            pltpu.SemaphoreType.DMA((WAVES, 2)),
            pltpu.SemaphoreType.DMA((WAVES, 3)),
            pltpu.SemaphoreType.DMA((WAVES, 4)),
            pltpu.SemaphoreType.REGULAR,
            pltpu.SemaphoreType.REGULAR,
        ],
        compiler_params=pltpu.CompilerParams(collective_id=0),
    )(dy, W)
